# baseline (speedup 1.0000x reference)
import jax
import jax.numpy as jnp
from jax.experimental import pallas as pl

NUM_HEADS = 8
OUT_FEATS = 64
ROW_BLOCK = 2000


def _proj_kernel(x_ref, w1_ref, w2_ref, o1_ref, o2_ref):
    x = x_ref[:]
    o1_ref[:] = jnp.dot(x, w1_ref[:], preferred_element_type=jnp.float32).astype(jnp.bfloat16)
    o2_ref[:] = jnp.dot(x, w2_ref[:], preferred_element_type=jnp.float32).astype(jnp.bfloat16)


def kernel(feat, edge_index, W_fc_self):
    del edge_index
    n, in_feats = feat.shape
    m = W_fc_self.shape[0]
    half = m // 2
    wt = W_fc_self.T
    o1, o2 = pl.pallas_call(
        _proj_kernel,
        grid=(n // ROW_BLOCK,),
        in_specs=[
            pl.BlockSpec((ROW_BLOCK, in_feats), lambda i: (i, 0)),
            pl.BlockSpec((in_feats, half), lambda i: (0, 0)),
            pl.BlockSpec((in_feats, half), lambda i: (0, 0)),
        ],
        out_specs=[
            pl.BlockSpec((ROW_BLOCK, half), lambda i: (i, 0)),
            pl.BlockSpec((ROW_BLOCK, half), lambda i: (i, 0)),
        ],
        out_shape=[
            jax.ShapeDtypeStruct((n, half), jnp.bfloat16),
            jax.ShapeDtypeStruct((n, half), jnp.bfloat16),
        ],
    )(feat, wt[:, :half], wt[:, half:])
    out = jnp.concatenate([o1, o2], axis=1)
    return out.astype(jnp.float32).reshape(n, NUM_HEADS, OUT_FEATS)
